# baseline (speedup 1.0000x reference)
import functools
import math

import jax
import jax.numpy as jnp
from jax import lax
from jax.experimental import pallas as pl
from jax.experimental.pallas import tpu as pltpu
from jax.experimental.pallas import tpu_sc as plsc

N = 100000
E = 3200000
G = 1024
D = 24
H = 128
NL = 3

NC = 2
NS = 16
NW = NC * NS
L = 16

_B = 128
_CHUNK = NW * _B
_EP = ((E + _CHUNK - 1) // _CHUNK) * _CHUNK
_ROWS = 8192

_INV_SQRT_D = 1.0 / math.sqrt(float(D))
_NEG_BIG = -3.0e38


def _linear_body(x_ref, w_ref, b_ref, o_ref):
    o_ref[...] = (
        jnp.dot(x_ref[...], w_ref[...], preferred_element_type=jnp.float32)
        + b_ref[...]
    )


def _linear(x, W, b, *, relu=False):
    n, d_in = x.shape
    d_out = W.shape[0]
    n_pad = (-n) % _ROWS
    xp = jnp.pad(x, ((0, n_pad), (0, 0))) if n_pad else x
    np_rows = xp.shape[0]
    out = pl.pallas_call(
        _linear_body,
        grid=(np_rows // _ROWS,),
        in_specs=[
            pl.BlockSpec((_ROWS, d_in), lambda i: (i, 0)),
            pl.BlockSpec((d_in, d_out), lambda i: (0, 0)),
            pl.BlockSpec((1, d_out), lambda i: (0, 0)),
        ],
        out_specs=pl.BlockSpec((_ROWS, d_out), lambda i: (i, 0)),
        out_shape=jax.ShapeDtypeStruct((np_rows, d_out), jnp.float32),
    )(xp, W.T, b[None, :])
    out = out[:n] if n_pad else out
    return jax.nn.relu(out) if relu else out


def _pass_a_body(qp_h, kp_h, dst_h, src_h, alpha_h, amaxp_h,
                 amax_v,
                 dsti0, srci0, qrows0, krows0, ab0,
                 dsti1, srci1, qrows1, krows1, ab1,
                 sem_i0, sem_i1, sem_r0, sem_r1, sem_w0, sem_w1):
    wid = lax.axis_index("s") * NC + lax.axis_index("c")
    per_tile = _EP // NW
    nblocks = per_tile // _B
    npairs = nblocks // 2

    bufs = ((dsti0, srci0, qrows0, krows0, ab0, sem_i0, sem_r0, sem_w0),
            (dsti1, srci1, qrows1, krows1, ab1, sem_i1, sem_r1, sem_w1))

    neg = jnp.full((L,), _NEG_BIG, jnp.float32)

    def init(i, c):
        amax_v[pl.ds(i * L, L)] = neg
        return c

    lax.fori_loop(0, (N + L) // L, init, 0)

    base = wid * per_tile
    e16 = lax.iota(jnp.int32, L)

    def idx_issue(b, k):
        d, s2, _, _, _, si, _, _ = bufs[k]
        pltpu.async_copy(dst_h.at[pl.ds(base + b * _B, _B)], d, si)
        pltpu.async_copy(src_h.at[pl.ds(base + b * _B, _B)], s2, si)

    def idx_wait(k):
        d, s2, _, _, _, si, _, _ = bufs[k]
        pltpu.make_async_copy(dst_h.at[pl.ds(0, _B)], d, si).wait()
        pltpu.make_async_copy(src_h.at[pl.ds(0, _B)], s2, si).wait()

    def rows_issue(k):
        d, s2, qr, kr, _, _, sr, _ = bufs[k]
        pltpu.async_copy(qp_h.at[d], qr, sr)
        pltpu.async_copy(kp_h.at[s2], kr, sr)

    def rows_wait(k):
        d, s2, qr, kr, _, _, sr, _ = bufs[k]
        pltpu.make_async_copy(qp_h.at[d], qr, sr).wait()
        pltpu.make_async_copy(kp_h.at[s2], kr, sr).wait()

    idx_issue(0, 0)
    idx_wait(0)
    rows_issue(0)
    idx_issue(1, 1)

    def pair(o, carry):
        for k in (0, 1):
            d, s2, qr, kr, ab, si, sr, sw = bufs[k]
            b = o * 2 + k
            rows_wait(k)

            @pl.when(o > 0)
            def _():
                pltpu.make_async_copy(ab, alpha_h.at[pl.ds(0, _B)], sw).wait()

            def grp(g, cc):
                ebase = g * L
                row = e16 + ebase

                def dot(jj, acc):
                    j0 = jj * 4
                    for t in range(4):
                        col = jnp.full((L,), 0, jnp.int32) + (j0 + t)
                        qv = plsc.load_gather(qr, [row, col])
                        kv = plsc.load_gather(kr, [row, col])
                        acc = acc + qv * kv
                    return acc

                acc = lax.fori_loop(0, D // 4, dot, jnp.zeros((L,), jnp.float32))
                acc = acc * _INV_SQRT_D
                ab[pl.ds(ebase, L)] = acc
                d16 = d[pl.ds(ebase, L)]
                cur = plsc.load_gather(amax_v, [d16])
                plsc.store_scatter(amax_v, [d16], jnp.maximum(cur, acc))
                return cc

            lax.fori_loop(0, _B // L, grp, 0)
            pltpu.async_copy(ab, alpha_h.at[pl.ds(base + b * _B, _B)], sw)

            @pl.when(b + 2 < nblocks)
            def _():
                idx_issue(b + 2, k)

            ko = 1 - k

            @pl.when(b + 1 < nblocks)
            def _():
                idx_wait(ko)
                rows_issue(ko)

        return carry

    lax.fori_loop(0, npairs, pair, 0)
    pltpu.make_async_copy(ab0, alpha_h.at[pl.ds(0, _B)], sem_w0).wait()
    pltpu.make_async_copy(ab1, alpha_h.at[pl.ds(0, _B)], sem_w1).wait()
    pltpu.sync_copy(amax_v.at[pl.ds(0, N)], amaxp_h.at[wid])


def _pass_a(qp, kp, dstp, srcp):
    mesh = plsc.VectorSubcoreMesh(core_axis_name="c", subcore_axis_name="s")
    f = pl.kernel(
        _pass_a_body,
        out_type=(
            jax.ShapeDtypeStruct((_EP,), jnp.float32),
            jax.ShapeDtypeStruct((NW, N), jnp.float32),
        ),
        mesh=mesh,
        compiler_params=pltpu.CompilerParams(
            needs_layout_passes=False, use_tc_tiling_on_sc=False
        ),
        scratch_types=[
            pltpu.VMEM((N + L,), jnp.float32),
            pltpu.VMEM((_B,), jnp.int32),
            pltpu.VMEM((_B,), jnp.int32),
            pltpu.VMEM((_B, 32), jnp.float32),
            pltpu.VMEM((_B, 32), jnp.float32),
            pltpu.VMEM((_B,), jnp.float32),
            pltpu.VMEM((_B,), jnp.int32),
            pltpu.VMEM((_B,), jnp.int32),
            pltpu.VMEM((_B, 32), jnp.float32),
            pltpu.VMEM((_B, 32), jnp.float32),
            pltpu.VMEM((_B,), jnp.float32),
            pltpu.SemaphoreType.DMA,
            pltpu.SemaphoreType.DMA,
            pltpu.SemaphoreType.DMA,
            pltpu.SemaphoreType.DMA,
            pltpu.SemaphoreType.DMA,
            pltpu.SemaphoreType.DMA,
        ],
    )
    return f(qp, kp, dstp, srcp)


def _ex_body(alpha_h, dst_h, amax_h, ex_h,
             amax_v, dsti0, ab0, exb0, dsti1, ab1, exb1,
             sem_i0, sem_i1, sem_w0, sem_w1):
    wid = lax.axis_index("s") * NC + lax.axis_index("c")
    per_tile = _EP // NW
    nblocks = per_tile // _B
    npairs = nblocks // 2
    bufs = ((dsti0, ab0, exb0, sem_i0, sem_w0),
            (dsti1, ab1, exb1, sem_i1, sem_w1))

    pltpu.sync_copy(amax_h, amax_v.at[pl.ds(0, N)])
    amax_v[pl.ds(N, L)] = jnp.zeros((L,), jnp.float32)

    base = wid * per_tile

    def idx_issue(b, k):
        d, ab, _, si, _ = bufs[k]
        off = base + b * _B
        pltpu.async_copy(dst_h.at[pl.ds(off, _B)], d, si)
        pltpu.async_copy(alpha_h.at[pl.ds(off, _B)], ab, si)

    def idx_wait(k):
        d, ab, _, si, _ = bufs[k]
        pltpu.make_async_copy(dst_h.at[pl.ds(0, _B)], d, si).wait()
        pltpu.make_async_copy(alpha_h.at[pl.ds(0, _B)], ab, si).wait()

    idx_issue(0, 0)
    idx_issue(1, 1)

    def pair(o, carry):
        for k in (0, 1):
            d, ab, exb, si, sw = bufs[k]
            b = o * 2 + k
            idx_wait(k)

            @pl.when(o > 0)
            def _():
                pltpu.make_async_copy(exb, ex_h.at[pl.ds(0, _B)], sw).wait()

            def grp(g, cc):
                ebase = g * L
                d16 = d[pl.ds(ebase, L)]
                am = plsc.load_gather(amax_v, [d16])
                exb[pl.ds(ebase, L)] = jnp.exp(ab[pl.ds(ebase, L)] - am)
                return cc

            lax.fori_loop(0, _B // L, grp, 0)
            pltpu.async_copy(exb, ex_h.at[pl.ds(base + b * _B, _B)], sw)

            @pl.when(b + 2 < nblocks)
            def _():
                idx_issue(b + 2, k)

        return carry

    lax.fori_loop(0, npairs, pair, 0)
    pltpu.make_async_copy(exb0, ex_h.at[pl.ds(0, _B)], sem_w0).wait()
    pltpu.make_async_copy(exb1, ex_h.at[pl.ds(0, _B)], sem_w1).wait()


def _ex_pass(alpha_p, dstp, amax):
    mesh = plsc.VectorSubcoreMesh(core_axis_name="c", subcore_axis_name="s")
    f = pl.kernel(
        _ex_body,
        out_type=jax.ShapeDtypeStruct((_EP,), jnp.float32),
        mesh=mesh,
        compiler_params=pltpu.CompilerParams(
            needs_layout_passes=False, use_tc_tiling_on_sc=False
        ),
        scratch_types=[
            pltpu.VMEM((N + L,), jnp.float32),
            pltpu.VMEM((_B,), jnp.int32),
            pltpu.VMEM((_B,), jnp.float32),
            pltpu.VMEM((_B,), jnp.float32),
            pltpu.VMEM((_B,), jnp.int32),
            pltpu.VMEM((_B,), jnp.float32),
            pltpu.VMEM((_B,), jnp.float32),
            pltpu.SemaphoreType.DMA,
            pltpu.SemaphoreType.DMA,
            pltpu.SemaphoreType.DMA,
            pltpu.SemaphoreType.DMA,
        ],
    )
    return f(alpha_p, dstp, amax)


_UROWS = 102400


def _pass_b_body(v0_h, v1_h, dst_h, src_h, ex_h, u0_h, u1_h,
                 u_sh, zbuf,
                 dsti0, srci0, exbb0, vrows0, wbuf0, dscat0,
                 dsti1, srci1, exbb1, vrows1, wbuf1, dscat1,
                 sem_i0, sem_i1, sem_r0, sem_r1,
                 sem_u0, sem_u1):
    c = lax.axis_index("c")
    s = lax.axis_index("s")
    per_tile = _EP // NS
    nblocks = per_tile // _B
    npairs = nblocks // 2
    rows_per_tile = _UROWS // NS

    bufs = ((dsti0, srci0, exbb0, vrows0, wbuf0, dscat0,
             sem_i0, sem_r0, sem_u0),
            (dsti1, srci1, exbb1, vrows1, wbuf1, dscat1,
             sem_i1, sem_r1, sem_u1))

    def zb(r, cc):
        zbuf[r, :] = jnp.zeros((L,), jnp.float32)
        return cc

    lax.fori_loop(0, 128, zb, 0)

    def z(i, cc):
        pltpu.sync_copy(zbuf, u_sh.at[pl.ds(s * rows_per_tile + i * 128, 128), :])
        return cc

    lax.fori_loop(0, rows_per_tile // 128, z, 0)

    plsc.subcore_barrier()

    base = s * per_tile
    e16 = lax.iota(jnp.int32, L)

    def idx_issue(b, k):
        d, s2, al, _, _, _, si, _, _ = bufs[k]
        off = base + b * _B
        pltpu.async_copy(dst_h.at[pl.ds(off, _B)], d, si)
        pltpu.async_copy(src_h.at[pl.ds(off, _B)], s2, si)
        pltpu.async_copy(ex_h.at[pl.ds(off, _B)], al, si)

    def idx_wait(k):
        d, s2, al, _, _, _, si, _, _ = bufs[k]
        pltpu.make_async_copy(dst_h.at[pl.ds(0, _B)], d, si).wait()
        pltpu.make_async_copy(src_h.at[pl.ds(0, _B)], s2, si).wait()
        pltpu.make_async_copy(ex_h.at[pl.ds(0, _B)], al, si).wait()

    def gath_issue(k):
        d, s2, _, vr, _, _, _, sr, _ = bufs[k]

        @pl.when(c == 0)
        def _():
            pltpu.async_copy(v0_h.at[s2], vr, sr)

        @pl.when(c != 0)
        def _():
            pltpu.async_copy(v1_h.at[s2], vr, sr)

    def gath_wait(k):
        d, s2, _, vr, _, _, _, sr, _ = bufs[k]
        pltpu.make_async_copy(v0_h.at[s2], vr, sr).wait()

    idx_issue(0, 0)
    idx_wait(0)
    gath_issue(0)
    idx_issue(1, 1)

    def pair(o, carry):
        for k in (0, 1):
            d, s2, al, vr, wb, dsc, si, sr, su = bufs[k]
            b = o * 2 + k
            gath_wait(k)

            @pl.when(o > 0)
            def _():
                pltpu.make_async_copy(wb, u_sh.at[d], su).wait()

            def grp(g, gg):
                ebase = g * L
                row = e16 + ebase
                dd = d[pl.ds(ebase, L)]
                dsc[pl.ds(ebase, L)] = dd
                ex = al[pl.ds(ebase, L)]

                def wcol(ff, cc2):
                    f0 = ff * 4
                    for t in range(4):
                        col = jnp.full((L,), 0, jnp.int32) + (f0 + t)
                        w = plsc.load_gather(vr, [row, col]) * ex
                        plsc.store_scatter(wb, [row, col], w)
                    return cc2

                lax.fori_loop(0, L // 4, wcol, 0)
                return gg

            lax.fori_loop(0, _B // L, grp, 0)
            pltpu.async_copy(wb, u_sh.at[dsc], su, add=True)

            @pl.when(b + 2 < nblocks)
            def _():
                idx_issue(b + 2, k)

            ko = 1 - k

            @pl.when(b + 1 < nblocks)
            def _():
                idx_wait(ko)
                gath_issue(ko)

        return carry

    lax.fori_loop(0, npairs, pair, 0)
    pltpu.make_async_copy(wbuf0, u_sh.at[dscat0], sem_u0).wait()
    pltpu.make_async_copy(wbuf1, u_sh.at[dscat1], sem_u1).wait()
    plsc.subcore_barrier()

    @pl.when(c == 0)
    def _():
        pltpu.sync_copy(
            u_sh.at[pl.ds(s * rows_per_tile, rows_per_tile), :],
            u0_h.at[pl.ds(s * rows_per_tile, rows_per_tile), :],
        )

    @pl.when(c != 0)
    def _():
        pltpu.sync_copy(
            u_sh.at[pl.ds(s * rows_per_tile, rows_per_tile), :],
            u1_h.at[pl.ds(s * rows_per_tile, rows_per_tile), :],
        )


def _pass_b(v0p, v1p, dstp, srcp, exd):
    mesh = plsc.VectorSubcoreMesh(core_axis_name="c", subcore_axis_name="s")
    f = pl.kernel(
        _pass_b_body,
        out_type=(
            jax.ShapeDtypeStruct((_UROWS, L), jnp.float32),
            jax.ShapeDtypeStruct((_UROWS, L), jnp.float32),
        ),
        mesh=mesh,
        compiler_params=pltpu.CompilerParams(
            needs_layout_passes=False, use_tc_tiling_on_sc=False
        ),
        scratch_types=[
            pltpu.VMEM_SHARED((_UROWS, L), jnp.float32),
            pltpu.VMEM((128, L), jnp.float32),
            pltpu.VMEM((_B,), jnp.int32),
            pltpu.VMEM((_B,), jnp.int32),
            pltpu.VMEM((_B,), jnp.float32),
            pltpu.VMEM((_B, L), jnp.float32),
            pltpu.VMEM((_B, L), jnp.float32),
            pltpu.VMEM((_B,), jnp.int32),
            pltpu.VMEM((_B,), jnp.int32),
            pltpu.VMEM((_B,), jnp.int32),
            pltpu.VMEM((_B,), jnp.float32),
            pltpu.VMEM((_B, L), jnp.float32),
            pltpu.VMEM((_B, L), jnp.float32),
            pltpu.VMEM((_B,), jnp.int32),
            pltpu.SemaphoreType.DMA,
            pltpu.SemaphoreType.DMA,
            pltpu.SemaphoreType.DMA,
            pltpu.SemaphoreType.DMA,
            pltpu.SemaphoreType.DMA,
            pltpu.SemaphoreType.DMA,
        ],
    )
    return f(v0p, v1p, dstp, srcp, exd)


def _amax_merge_body(p_ref, o_ref):
    o_ref[...] = jnp.max(p_ref[...], axis=0, keepdims=True)


def _amax_merge(amaxp):
    return pl.pallas_call(
        _amax_merge_body,
        out_shape=jax.ShapeDtypeStruct((1, N), jnp.float32),
    )(amaxp)[0]


_PRJ = 2048


def _project_body(x_ref, wq_ref, wk_ref, w0_ref, w1_ref, ws_ref,
                  qp_ref, kp_ref, v0_ref, v1_ref, s_ref):
    x = x_ref[...]
    dot = lambda w: jnp.dot(x, w[...], preferred_element_type=jnp.float32)
    qp_ref[...] = dot(wq_ref)
    kp_ref[...] = dot(wk_ref)
    v0_ref[...] = dot(w0_ref)
    v1_ref[...] = dot(w1_ref)
    s_ref[...] = dot(ws_ref)


def _project(h_pad, params, i):
    def wpad(W, b, cols, col0=0, one_col=None):
        m = jnp.zeros((D + 1, cols), jnp.float32)
        m = lax.dynamic_update_slice(m, W.T[:, col0:col0 + 12] if cols == L else W.T, (0, 0))
        m = lax.dynamic_update_slice(m, (b[col0:col0 + 12] if cols == L else b)[None, :], (D, 0))
        if one_col is not None:
            m = m.at[D, one_col].set(1.0)
        return m

    Wq = wpad(params["conv%d_Wq" % i], params["conv%d_bq" % i], 32)
    Wk = wpad(params["conv%d_Wk" % i], params["conv%d_bk" % i], 32)
    Wv = params["conv%d_Wv" % i]
    bv = params["conv%d_bv" % i]
    W0 = jnp.zeros((D + 1, L), jnp.float32)
    W0 = lax.dynamic_update_slice(W0, Wv.T[:, :12], (0, 0))
    W0 = lax.dynamic_update_slice(W0, bv[None, :12], (D, 0))
    W0 = W0.at[D, 12].set(1.0)
    W1 = jnp.zeros((D + 1, L), jnp.float32)
    W1 = lax.dynamic_update_slice(W1, Wv.T[:, 12:], (0, 0))
    W1 = lax.dynamic_update_slice(W1, bv[None, 12:], (D, 0))
    Ws = jnp.concatenate(
        [params["conv%d_Ws" % i].T, params["conv%d_bs" % i][None, :]], axis=0
    )
    return pl.pallas_call(
        _project_body,
        grid=(_NPAD // _PRJ,),
        in_specs=[
            pl.BlockSpec((_PRJ, D + 1), lambda j: (j, 0)),
            pl.BlockSpec((D + 1, 32), lambda j: (0, 0)),
            pl.BlockSpec((D + 1, 32), lambda j: (0, 0)),
            pl.BlockSpec((D + 1, L), lambda j: (0, 0)),
            pl.BlockSpec((D + 1, L), lambda j: (0, 0)),
            pl.BlockSpec((D + 1, D), lambda j: (0, 0)),
        ],
        out_specs=[
            pl.BlockSpec((_PRJ, 32), lambda j: (j, 0)),
            pl.BlockSpec((_PRJ, 32), lambda j: (j, 0)),
            pl.BlockSpec((_PRJ, L), lambda j: (j, 0)),
            pl.BlockSpec((_PRJ, L), lambda j: (j, 0)),
            pl.BlockSpec((_PRJ, D), lambda j: (j, 0)),
        ],
        out_shape=[
            jax.ShapeDtypeStruct((_NPAD, 32), jnp.float32),
            jax.ShapeDtypeStruct((_NPAD, 32), jnp.float32),
            jax.ShapeDtypeStruct((_NPAD, L), jnp.float32),
            jax.ShapeDtypeStruct((_NPAD, L), jnp.float32),
            jax.ShapeDtypeStruct((_NPAD, D), jnp.float32),
        ],
    )(h_pad, Wq, Wk, W0, W1, Ws)


def _combine_body(u0_ref, u1_ref, s_ref, h_ref):
    u0 = u0_ref[...]
    u1 = u1_ref[...]
    den = u0[:, 12:13] + 1e-16
    num = jnp.concatenate([u0[:, :12], u1[:, :12]], axis=1)
    h = jnp.maximum(num / den + s_ref[...], 0.0)
    ones = jnp.ones((h.shape[0], 1), jnp.float32)
    h_ref[...] = jnp.concatenate([h, ones], axis=1)


def _combine(u0, u1, s_pad):
    return pl.pallas_call(
        _combine_body,
        grid=(_NPAD // _PRJ,),
        in_specs=[
            pl.BlockSpec((_PRJ, L), lambda j: (j, 0)),
            pl.BlockSpec((_PRJ, L), lambda j: (j, 0)),
            pl.BlockSpec((_PRJ, D), lambda j: (j, 0)),
        ],
        out_specs=pl.BlockSpec((_PRJ, D + 1), lambda j: (j, 0)),
        out_shape=jax.ShapeDtypeStruct((_NPAD, D + 1), jnp.float32),
    )(u0, u1, s_pad)


_PR = 2048
_NPAD = 102400


def _pool_body(b_ref, h_ref, o_ref):
    i = pl.program_id(0)

    @pl.when(i == 0)
    def _():
        o_ref[...] = jnp.zeros_like(o_ref)

    ids = b_ref[0, :]
    onehot = (ids[:, None] == lax.broadcasted_iota(jnp.int32, (1, G), 1)).astype(
        jnp.float32
    )
    o_ref[...] += lax.dot_general(
        onehot,
        h_ref[...],
        (((0,), (0,)), ((), ())),
        preferred_element_type=jnp.float32,
    )


def _pool(h_pad, batch):
    bpad = jnp.full((1, _NPAD), G, jnp.int32)
    bpad = lax.dynamic_update_slice(bpad, batch[None, :], (0, 0))
    h_aug = jnp.pad(h_pad, ((0, 0), (0, 32 - (D + 1))))
    return pl.pallas_call(
        _pool_body,
        grid=(_NPAD // _PR,),
        in_specs=[
            pl.BlockSpec((1, _PR), lambda i: (0, i)),
            pl.BlockSpec((_PR, 32), lambda i: (i, 0)),
        ],
        out_specs=pl.BlockSpec((G, 32), lambda i: (0, 0)),
        out_shape=jax.ShapeDtypeStruct((G, 32), jnp.float32),
    )(bpad, h_aug)


def _pad_rows(a, cols=32, extra_rows=L):
    n, d = a.shape
    out = jnp.zeros((n + extra_rows, cols), jnp.float32)
    return lax.dynamic_update_slice(out, a, (0, 0))


def kernel(x, edge_index, batch, global_features, params):
    src = edge_index[0]
    dst = edge_index[1]
    pad = jnp.full((_EP - E,), N, jnp.int32)
    srcp = jnp.concatenate([src, pad])
    dstp = jnp.concatenate([dst, pad])

    h_pad = jnp.zeros((_NPAD, D + 1), jnp.float32)
    h_pad = lax.dynamic_update_slice(h_pad, x, (0, 0))
    h_pad = h_pad.at[:, D].set(1.0)
    for i in range(NL):
        qp, kp, v0p, v1p, s_pad = _project(h_pad, params, i)
        alpha_p, amaxp = _pass_a(qp, kp, dstp, srcp)
        amax = _amax_merge(amaxp)
        exd = _ex_pass(alpha_p, dstp, amax)
        u0, u1 = _pass_b(v0p, v1p, dstp, srcp, exd)
        h_pad = _combine(u0, u1, s_pad)

    pooled_aug = _pool(h_pad, batch)
    pooled = pooled_aug[:, :D] / jnp.maximum(pooled_aug[:, D:D + 1], 1.0)
    z = _linear(pooled, params["lin1_W"], params["lin1_b"], relu=True)
    z = _linear(z, params["lin2_W"], params["lin2_b"], relu=True)
    z = _linear(z, params["lin3_W"], params["lin3_b"])
    return jnp.squeeze(z, axis=-1)


# speedup vs baseline: 1.1561x; 1.1561x over previous
import functools
import math

import jax
import jax.numpy as jnp
from jax import lax
from jax.experimental import pallas as pl
from jax.experimental.pallas import tpu as pltpu
from jax.experimental.pallas import tpu_sc as plsc

N = 100000
E = 3200000
G = 1024
D = 24
H = 128
NL = 3

NC = 2
NS = 16
NW = NC * NS
L = 16

_B = 128
_CHUNK = NW * _B
_EP = ((E + _CHUNK - 1) // _CHUNK) * _CHUNK
_ROWS = 8192

_INV_SQRT_D = 1.0 / math.sqrt(float(D))
_NEG_BIG = -3.0e38


def _linear_body(x_ref, w_ref, b_ref, o_ref):
    o_ref[...] = (
        jnp.dot(x_ref[...], w_ref[...], preferred_element_type=jnp.float32)
        + b_ref[...]
    )


def _linear(x, W, b, *, relu=False):
    n, d_in = x.shape
    d_out = W.shape[0]
    n_pad = (-n) % _ROWS
    xp = jnp.pad(x, ((0, n_pad), (0, 0))) if n_pad else x
    np_rows = xp.shape[0]
    out = pl.pallas_call(
        _linear_body,
        grid=(np_rows // _ROWS,),
        in_specs=[
            pl.BlockSpec((_ROWS, d_in), lambda i: (i, 0)),
            pl.BlockSpec((d_in, d_out), lambda i: (0, 0)),
            pl.BlockSpec((1, d_out), lambda i: (0, 0)),
        ],
        out_specs=pl.BlockSpec((_ROWS, d_out), lambda i: (i, 0)),
        out_shape=jax.ShapeDtypeStruct((np_rows, d_out), jnp.float32),
    )(xp, W.T, b[None, :])
    out = out[:n] if n_pad else out
    return jax.nn.relu(out) if relu else out


_BA = 64


def _pass_a_body(qp_h, kp_h, dst_h, src_h, alpha_h, amaxp_h,
                 amax_v,
                 dsti0, srci0, qrows0, krows0, ab0,
                 dsti1, srci1, qrows1, krows1, ab1,
                 dsti2, srci2, qrows2, krows2, ab2,
                 dsti3, srci3, qrows3, krows3, ab3,
                 sem_i0, sem_i1, sem_i2, sem_i3,
                 sem_r0, sem_r1, sem_r2, sem_r3,
                 sem_w0, sem_w1, sem_w2, sem_w3):
    wid = lax.axis_index("s") * NC + lax.axis_index("c")
    per_tile = _EP // NW
    nblocks = per_tile // _BA
    nquads = nblocks // 4

    bufs = ((dsti0, srci0, qrows0, krows0, ab0, sem_i0, sem_r0, sem_w0),
            (dsti1, srci1, qrows1, krows1, ab1, sem_i1, sem_r1, sem_w1),
            (dsti2, srci2, qrows2, krows2, ab2, sem_i2, sem_r2, sem_w2),
            (dsti3, srci3, qrows3, krows3, ab3, sem_i3, sem_r3, sem_w3))

    neg = jnp.full((L,), _NEG_BIG, jnp.float32)

    def init(i, c):
        amax_v[pl.ds(i * L, L)] = neg
        return c

    lax.fori_loop(0, (N + L) // L, init, 0)

    base = wid * per_tile
    e16 = lax.iota(jnp.int32, L)

    def idx_issue(b, k):
        d, s2, _, _, _, si, _, _ = bufs[k]
        pltpu.async_copy(dst_h.at[pl.ds(base + b * _BA, _BA)], d, si)
        pltpu.async_copy(src_h.at[pl.ds(base + b * _BA, _BA)], s2, si)

    def idx_wait(k):
        d, s2, _, _, _, si, _, _ = bufs[k]
        pltpu.make_async_copy(dst_h.at[pl.ds(0, _BA)], d, si).wait()
        pltpu.make_async_copy(src_h.at[pl.ds(0, _BA)], s2, si).wait()

    def rows_issue(k):
        d, s2, qr, kr, _, _, sr, _ = bufs[k]
        pltpu.async_copy(qp_h.at[d], qr, sr)
        pltpu.async_copy(kp_h.at[s2], kr, sr)

    def rows_wait(k):
        d, s2, qr, kr, _, _, sr, _ = bufs[k]
        pltpu.make_async_copy(qp_h.at[d], qr, sr).wait()
        pltpu.make_async_copy(kp_h.at[s2], kr, sr).wait()

    for j in range(4):
        idx_issue(j, j)
    idx_wait(0)
    rows_issue(0)
    idx_wait(1)
    rows_issue(1)

    def quad(o, carry):
        for k in range(4):
            d, s2, qr, kr, ab, si, sr, sw = bufs[k]
            b = o * 4 + k
            rows_wait(k)

            @pl.when(o > 0)
            def _():
                pltpu.make_async_copy(ab, alpha_h.at[pl.ds(0, _BA)], sw).wait()

            def grp(g, cc):
                ebase = g * L
                row = e16 + ebase

                def dot(jj, acc):
                    j0 = jj * 4
                    for t in range(4):
                        col = jnp.full((L,), 0, jnp.int32) + (j0 + t)
                        qv = plsc.load_gather(qr, [row, col])
                        kv = plsc.load_gather(kr, [row, col])
                        acc = acc + qv * kv
                    return acc

                acc = lax.fori_loop(0, D // 4, dot, jnp.zeros((L,), jnp.float32))
                acc = acc * _INV_SQRT_D
                ab[pl.ds(ebase, L)] = acc
                d16 = d[pl.ds(ebase, L)]
                cur = plsc.load_gather(amax_v, [d16])
                plsc.store_scatter(amax_v, [d16], jnp.maximum(cur, acc))
                return cc

            lax.fori_loop(0, _BA // L, grp, 0)
            pltpu.async_copy(ab, alpha_h.at[pl.ds(base + b * _BA, _BA)], sw)

            @pl.when(b + 4 < nblocks)
            def _():
                idx_issue(b + 4, k)

            k2 = (k + 2) % 4

            @pl.when(b + 2 < nblocks)
            def _():
                idx_wait(k2)
                rows_issue(k2)

        return carry

    lax.fori_loop(0, nquads, quad, 0)
    pltpu.make_async_copy(ab0, alpha_h.at[pl.ds(0, _BA)], sem_w0).wait()
    pltpu.make_async_copy(ab1, alpha_h.at[pl.ds(0, _BA)], sem_w1).wait()
    pltpu.make_async_copy(ab2, alpha_h.at[pl.ds(0, _BA)], sem_w2).wait()
    pltpu.make_async_copy(ab3, alpha_h.at[pl.ds(0, _BA)], sem_w3).wait()
    pltpu.sync_copy(amax_v.at[pl.ds(0, N)], amaxp_h.at[wid])


def _pass_a(qp, kp, dstp, srcp):
    mesh = plsc.VectorSubcoreMesh(core_axis_name="c", subcore_axis_name="s")
    f = pl.kernel(
        _pass_a_body,
        out_type=(
            jax.ShapeDtypeStruct((_EP,), jnp.float32),
            jax.ShapeDtypeStruct((NW, N), jnp.float32),
        ),
        mesh=mesh,
        compiler_params=pltpu.CompilerParams(
            needs_layout_passes=False, use_tc_tiling_on_sc=False
        ),
        scratch_types=[
            pltpu.VMEM((N + L,), jnp.float32),
            pltpu.VMEM((_BA,), jnp.int32),
            pltpu.VMEM((_BA,), jnp.int32),
            pltpu.VMEM((_BA, 32), jnp.float32),
            pltpu.VMEM((_BA, 32), jnp.float32),
            pltpu.VMEM((_BA,), jnp.float32),
            pltpu.VMEM((_BA,), jnp.int32),
            pltpu.VMEM((_BA,), jnp.int32),
            pltpu.VMEM((_BA, 32), jnp.float32),
            pltpu.VMEM((_BA, 32), jnp.float32),
            pltpu.VMEM((_BA,), jnp.float32),
            pltpu.VMEM((_BA,), jnp.int32),
            pltpu.VMEM((_BA,), jnp.int32),
            pltpu.VMEM((_BA, 32), jnp.float32),
            pltpu.VMEM((_BA, 32), jnp.float32),
            pltpu.VMEM((_BA,), jnp.float32),
            pltpu.VMEM((_BA,), jnp.int32),
            pltpu.VMEM((_BA,), jnp.int32),
            pltpu.VMEM((_BA, 32), jnp.float32),
            pltpu.VMEM((_BA, 32), jnp.float32),
            pltpu.VMEM((_BA,), jnp.float32),
            pltpu.SemaphoreType.DMA,
            pltpu.SemaphoreType.DMA,
            pltpu.SemaphoreType.DMA,
            pltpu.SemaphoreType.DMA,
            pltpu.SemaphoreType.DMA,
            pltpu.SemaphoreType.DMA,
            pltpu.SemaphoreType.DMA,
            pltpu.SemaphoreType.DMA,
            pltpu.SemaphoreType.DMA,
            pltpu.SemaphoreType.DMA,
            pltpu.SemaphoreType.DMA,
            pltpu.SemaphoreType.DMA,
        ],
    )
    return f(qp, kp, dstp, srcp)


_UROWS = 102400


def _pass_b_body(v0_h, v1_h, dst_h, src_h, alpha_h, amax_h, u0_h, u1_h,
                 u_sh, amax_sh, zbuf,
                 dsti0, srci0, alphab0, amb0, vrows0, wbuf0, dscat0,
                 dsti1, srci1, alphab1, amb1, vrows1, wbuf1, dscat1,
                 sem_i0, sem_i1, sem_r0, sem_r1, sem_a0, sem_a1,
                 sem_u0, sem_u1):
    c = lax.axis_index("c")
    s = lax.axis_index("s")
    per_tile = _EP // NS
    nblocks = per_tile // _B
    npairs = nblocks // 2
    rows_per_tile = _UROWS // NS

    bufs = ((dsti0, srci0, alphab0, amb0, vrows0, wbuf0, dscat0,
             sem_i0, sem_r0, sem_a0, sem_u0),
            (dsti1, srci1, alphab1, amb1, vrows1, wbuf1, dscat1,
             sem_i1, sem_r1, sem_a1, sem_u1))

    def zb(r, cc):
        zbuf[r, :] = jnp.zeros((L,), jnp.float32)
        return cc

    lax.fori_loop(0, 128, zb, 0)

    def z(i, cc):
        pltpu.sync_copy(zbuf, u_sh.at[pl.ds(s * rows_per_tile + i * 128, 128), :])
        return cc

    lax.fori_loop(0, rows_per_tile // 128, z, 0)

    @pl.when(s == 0)
    def _():
        pltpu.sync_copy(amax_h, amax_sh.at[pl.ds(0, N)])
        pltpu.sync_copy(zbuf.at[0, :], amax_sh.at[pl.ds(N, L)])

    plsc.subcore_barrier()

    base = s * per_tile
    e16 = lax.iota(jnp.int32, L)

    def idx_issue(b, k):
        d, s2, al, _, _, _, _, si, _, _, _ = bufs[k]
        off = base + b * _B
        pltpu.async_copy(dst_h.at[pl.ds(off, _B)], d, si)
        pltpu.async_copy(src_h.at[pl.ds(off, _B)], s2, si)
        pltpu.async_copy(alpha_h.at[pl.ds(off, _B)], al, si)

    def idx_wait(k):
        d, s2, al, _, _, _, _, si, _, _, _ = bufs[k]
        pltpu.make_async_copy(dst_h.at[pl.ds(0, _B)], d, si).wait()
        pltpu.make_async_copy(src_h.at[pl.ds(0, _B)], s2, si).wait()
        pltpu.make_async_copy(alpha_h.at[pl.ds(0, _B)], al, si).wait()

    def gath_issue(k):
        d, s2, _, am, vr, _, _, _, sr, sa, _ = bufs[k]
        pltpu.async_copy(amax_sh.at[d], am, sa)

        @pl.when(c == 0)
        def _():
            pltpu.async_copy(v0_h.at[s2], vr, sr)

        @pl.when(c != 0)
        def _():
            pltpu.async_copy(v1_h.at[s2], vr, sr)

    def gath_wait(k):
        d, s2, _, am, vr, _, _, _, sr, sa, _ = bufs[k]
        pltpu.make_async_copy(amax_sh.at[d], am, sa).wait()
        pltpu.make_async_copy(v0_h.at[s2], vr, sr).wait()

    idx_issue(0, 0)
    idx_wait(0)
    gath_issue(0)
    idx_issue(1, 1)

    def pair(o, carry):
        for k in (0, 1):
            d, s2, al, am, vr, wb, dsc, si, sr, sa, su = bufs[k]
            b = o * 2 + k
            gath_wait(k)

            @pl.when(o > 0)
            def _():
                pltpu.make_async_copy(wb, u_sh.at[d], su).wait()

            def grp(g, gg):
                ebase = g * L
                row = e16 + ebase
                dd = d[pl.ds(ebase, L)]
                dsc[pl.ds(ebase, L)] = dd
                ex = jnp.exp(al[pl.ds(ebase, L)] - am[pl.ds(ebase, L)])

                def wcol(ff, cc2):
                    f0 = ff * 4
                    for t in range(4):
                        col = jnp.full((L,), 0, jnp.int32) + (f0 + t)
                        w = plsc.load_gather(vr, [row, col]) * ex
                        plsc.store_scatter(wb, [row, col], w)
                    return cc2

                lax.fori_loop(0, L // 4, wcol, 0)
                return gg

            lax.fori_loop(0, _B // L, grp, 0)
            pltpu.async_copy(wb, u_sh.at[dsc], su, add=True)

            @pl.when(b + 2 < nblocks)
            def _():
                idx_issue(b + 2, k)

            ko = 1 - k

            @pl.when(b + 1 < nblocks)
            def _():
                idx_wait(ko)
                gath_issue(ko)

        return carry

    lax.fori_loop(0, npairs, pair, 0)
    pltpu.make_async_copy(wbuf0, u_sh.at[dscat0], sem_u0).wait()
    pltpu.make_async_copy(wbuf1, u_sh.at[dscat1], sem_u1).wait()
    plsc.subcore_barrier()

    @pl.when(c == 0)
    def _():
        pltpu.sync_copy(
            u_sh.at[pl.ds(s * rows_per_tile, rows_per_tile), :],
            u0_h.at[pl.ds(s * rows_per_tile, rows_per_tile), :],
        )

    @pl.when(c != 0)
    def _():
        pltpu.sync_copy(
            u_sh.at[pl.ds(s * rows_per_tile, rows_per_tile), :],
            u1_h.at[pl.ds(s * rows_per_tile, rows_per_tile), :],
        )


def _pass_b(v0p, v1p, dstp, srcp, alpha_p, amax):
    mesh = plsc.VectorSubcoreMesh(core_axis_name="c", subcore_axis_name="s")
    f = pl.kernel(
        _pass_b_body,
        out_type=(
            jax.ShapeDtypeStruct((_UROWS, L), jnp.float32),
            jax.ShapeDtypeStruct((_UROWS, L), jnp.float32),
        ),
        mesh=mesh,
        compiler_params=pltpu.CompilerParams(
            needs_layout_passes=False, use_tc_tiling_on_sc=False
        ),
        scratch_types=[
            pltpu.VMEM_SHARED((_UROWS, L), jnp.float32),
            pltpu.VMEM_SHARED((N + L,), jnp.float32),
            pltpu.VMEM((128, L), jnp.float32),
            pltpu.VMEM((_B,), jnp.int32),
            pltpu.VMEM((_B,), jnp.int32),
            pltpu.VMEM((_B,), jnp.float32),
            pltpu.VMEM((_B,), jnp.float32),
            pltpu.VMEM((_B, L), jnp.float32),
            pltpu.VMEM((_B, L), jnp.float32),
            pltpu.VMEM((_B,), jnp.int32),
            pltpu.VMEM((_B,), jnp.int32),
            pltpu.VMEM((_B,), jnp.int32),
            pltpu.VMEM((_B,), jnp.float32),
            pltpu.VMEM((_B,), jnp.float32),
            pltpu.VMEM((_B, L), jnp.float32),
            pltpu.VMEM((_B, L), jnp.float32),
            pltpu.VMEM((_B,), jnp.int32),
            pltpu.SemaphoreType.DMA,
            pltpu.SemaphoreType.DMA,
            pltpu.SemaphoreType.DMA,
            pltpu.SemaphoreType.DMA,
            pltpu.SemaphoreType.DMA,
            pltpu.SemaphoreType.DMA,
            pltpu.SemaphoreType.DMA,
            pltpu.SemaphoreType.DMA,
        ],
    )
    return f(v0p, v1p, dstp, srcp, alpha_p, amax)


def _amax_merge_body(p_ref, o_ref):
    o_ref[...] = jnp.max(p_ref[...], axis=0, keepdims=True)


def _amax_merge(amaxp):
    return pl.pallas_call(
        _amax_merge_body,
        out_shape=jax.ShapeDtypeStruct((1, N), jnp.float32),
    )(amaxp)[0]


_PRJ = 2048


def _project_body(x_ref, wq_ref, wk_ref, w0_ref, w1_ref, ws_ref,
                  qp_ref, kp_ref, v0_ref, v1_ref, s_ref):
    x = x_ref[...]
    dot = lambda w: jnp.dot(x, w[...], preferred_element_type=jnp.float32)
    qp_ref[...] = dot(wq_ref)
    kp_ref[...] = dot(wk_ref)
    v0_ref[...] = dot(w0_ref)
    v1_ref[...] = dot(w1_ref)
    s_ref[...] = dot(ws_ref)


def _project(h_pad, params, i):
    def wpad(W, b, cols, col0=0, one_col=None):
        m = jnp.zeros((D + 1, cols), jnp.float32)
        m = lax.dynamic_update_slice(m, W.T[:, col0:col0 + 12] if cols == L else W.T, (0, 0))
        m = lax.dynamic_update_slice(m, (b[col0:col0 + 12] if cols == L else b)[None, :], (D, 0))
        if one_col is not None:
            m = m.at[D, one_col].set(1.0)
        return m

    Wq = wpad(params["conv%d_Wq" % i], params["conv%d_bq" % i], 32)
    Wk = wpad(params["conv%d_Wk" % i], params["conv%d_bk" % i], 32)
    Wv = params["conv%d_Wv" % i]
    bv = params["conv%d_bv" % i]
    W0 = jnp.zeros((D + 1, L), jnp.float32)
    W0 = lax.dynamic_update_slice(W0, Wv.T[:, :12], (0, 0))
    W0 = lax.dynamic_update_slice(W0, bv[None, :12], (D, 0))
    W0 = W0.at[D, 12].set(1.0)
    W1 = jnp.zeros((D + 1, L), jnp.float32)
    W1 = lax.dynamic_update_slice(W1, Wv.T[:, 12:], (0, 0))
    W1 = lax.dynamic_update_slice(W1, bv[None, 12:], (D, 0))
    Ws = jnp.concatenate(
        [params["conv%d_Ws" % i].T, params["conv%d_bs" % i][None, :]], axis=0
    )
    return pl.pallas_call(
        _project_body,
        grid=(_NPAD // _PRJ,),
        in_specs=[
            pl.BlockSpec((_PRJ, D + 1), lambda j: (j, 0)),
            pl.BlockSpec((D + 1, 32), lambda j: (0, 0)),
            pl.BlockSpec((D + 1, 32), lambda j: (0, 0)),
            pl.BlockSpec((D + 1, L), lambda j: (0, 0)),
            pl.BlockSpec((D + 1, L), lambda j: (0, 0)),
            pl.BlockSpec((D + 1, D), lambda j: (0, 0)),
        ],
        out_specs=[
            pl.BlockSpec((_PRJ, 32), lambda j: (j, 0)),
            pl.BlockSpec((_PRJ, 32), lambda j: (j, 0)),
            pl.BlockSpec((_PRJ, L), lambda j: (j, 0)),
            pl.BlockSpec((_PRJ, L), lambda j: (j, 0)),
            pl.BlockSpec((_PRJ, D), lambda j: (j, 0)),
        ],
        out_shape=[
            jax.ShapeDtypeStruct((_NPAD, 32), jnp.float32),
            jax.ShapeDtypeStruct((_NPAD, 32), jnp.float32),
            jax.ShapeDtypeStruct((_NPAD, L), jnp.float32),
            jax.ShapeDtypeStruct((_NPAD, L), jnp.float32),
            jax.ShapeDtypeStruct((_NPAD, D), jnp.float32),
        ],
    )(h_pad, Wq, Wk, W0, W1, Ws)


def _combine_body(u0_ref, u1_ref, s_ref, h_ref):
    u0 = u0_ref[...]
    u1 = u1_ref[...]
    den = u0[:, 12:13] + 1e-16
    num = jnp.concatenate([u0[:, :12], u1[:, :12]], axis=1)
    h = jnp.maximum(num / den + s_ref[...], 0.0)
    ones = jnp.ones((h.shape[0], 1), jnp.float32)
    h_ref[...] = jnp.concatenate([h, ones], axis=1)


def _combine(u0, u1, s_pad):
    return pl.pallas_call(
        _combine_body,
        grid=(_NPAD // _PRJ,),
        in_specs=[
            pl.BlockSpec((_PRJ, L), lambda j: (j, 0)),
            pl.BlockSpec((_PRJ, L), lambda j: (j, 0)),
            pl.BlockSpec((_PRJ, D), lambda j: (j, 0)),
        ],
        out_specs=pl.BlockSpec((_PRJ, D + 1), lambda j: (j, 0)),
        out_shape=jax.ShapeDtypeStruct((_NPAD, D + 1), jnp.float32),
    )(u0, u1, s_pad)


_PR = 2048
_NPAD = 102400


def _pool_body(b_ref, h_ref, o_ref):
    i = pl.program_id(0)

    @pl.when(i == 0)
    def _():
        o_ref[...] = jnp.zeros_like(o_ref)

    ids = b_ref[0, :]
    onehot = (ids[:, None] == lax.broadcasted_iota(jnp.int32, (1, G), 1)).astype(
        jnp.float32
    )
    o_ref[...] += lax.dot_general(
        onehot,
        h_ref[...],
        (((0,), (0,)), ((), ())),
        preferred_element_type=jnp.float32,
    )


def _pool(h_pad, batch):
    bpad = jnp.full((1, _NPAD), G, jnp.int32)
    bpad = lax.dynamic_update_slice(bpad, batch[None, :], (0, 0))
    h_aug = jnp.pad(h_pad, ((0, 0), (0, 32 - (D + 1))))
    return pl.pallas_call(
        _pool_body,
        grid=(_NPAD // _PR,),
        in_specs=[
            pl.BlockSpec((1, _PR), lambda i: (0, i)),
            pl.BlockSpec((_PR, 32), lambda i: (i, 0)),
        ],
        out_specs=pl.BlockSpec((G, 32), lambda i: (0, 0)),
        out_shape=jax.ShapeDtypeStruct((G, 32), jnp.float32),
    )(bpad, h_aug)


def _pad_rows(a, cols=32, extra_rows=L):
    n, d = a.shape
    out = jnp.zeros((n + extra_rows, cols), jnp.float32)
    return lax.dynamic_update_slice(out, a, (0, 0))


def kernel(x, edge_index, batch, global_features, params):
    src = edge_index[0]
    dst = edge_index[1]
    pad = jnp.full((_EP - E,), N, jnp.int32)
    srcp = jnp.concatenate([src, pad])
    dstp = jnp.concatenate([dst, pad])

    h_pad = jnp.zeros((_NPAD, D + 1), jnp.float32)
    h_pad = lax.dynamic_update_slice(h_pad, x, (0, 0))
    h_pad = h_pad.at[:, D].set(1.0)
    for i in range(NL):
        qp, kp, v0p, v1p, s_pad = _project(h_pad, params, i)
        alpha_p, amaxp = _pass_a(qp, kp, dstp, srcp)
        amax = _amax_merge(amaxp)
        u0, u1 = _pass_b(v0p, v1p, dstp, srcp, alpha_p, amax)
        h_pad = _combine(u0, u1, s_pad)

    pooled_aug = _pool(h_pad, batch)
    pooled = pooled_aug[:, :D] / jnp.maximum(pooled_aug[:, D:D + 1], 1.0)
    z = _linear(pooled, params["lin1_W"], params["lin1_b"], relu=True)
    z = _linear(z, params["lin2_W"], params["lin2_b"], relu=True)
    z = _linear(z, params["lin3_W"], params["lin3_b"])
    return jnp.squeeze(z, axis=-1)


# speedup vs baseline: 1.3921x; 1.2042x over previous
import functools
import math

import jax
import jax.numpy as jnp
from jax import lax
from jax.experimental import pallas as pl
from jax.experimental.pallas import tpu as pltpu
from jax.experimental.pallas import tpu_sc as plsc

N = 100000
E = 3200000
G = 1024
D = 24
H = 128
NL = 3

NC = 2
NS = 16
NW = NC * NS
L = 16

_B = 128
_CHUNK = NW * _B
_EP = ((E + _CHUNK - 1) // _CHUNK) * _CHUNK
_ROWS = 8192

_INV_SQRT_D = 1.0 / math.sqrt(float(D))
_NEG_BIG = -3.0e38


def _linear_body(x_ref, w_ref, b_ref, o_ref):
    o_ref[...] = (
        jnp.dot(x_ref[...], w_ref[...], preferred_element_type=jnp.float32)
        + b_ref[...]
    )


def _linear(x, W, b, *, relu=False):
    n, d_in = x.shape
    d_out = W.shape[0]
    n_pad = (-n) % _ROWS
    xp = jnp.pad(x, ((0, n_pad), (0, 0))) if n_pad else x
    np_rows = xp.shape[0]
    out = pl.pallas_call(
        _linear_body,
        grid=(np_rows // _ROWS,),
        in_specs=[
            pl.BlockSpec((_ROWS, d_in), lambda i: (i, 0)),
            pl.BlockSpec((d_in, d_out), lambda i: (0, 0)),
            pl.BlockSpec((1, d_out), lambda i: (0, 0)),
        ],
        out_specs=pl.BlockSpec((_ROWS, d_out), lambda i: (i, 0)),
        out_shape=jax.ShapeDtypeStruct((np_rows, d_out), jnp.float32),
    )(xp, W.T, b[None, :])
    out = out[:n] if n_pad else out
    return jax.nn.relu(out) if relu else out


_BA = 64


def _pass_a_body(qp_h, kp_h, dst_h, src_h, alpha_h, amaxp_h,
                 amax_v,
                 dsti0, srci0, qrows0, krows0, ab0,
                 dsti1, srci1, qrows1, krows1, ab1,
                 dsti2, srci2, qrows2, krows2, ab2,
                 dsti3, srci3, qrows3, krows3, ab3,
                 sem_i0, sem_i1, sem_i2, sem_i3,
                 sem_r0, sem_r1, sem_r2, sem_r3,
                 sem_w0, sem_w1, sem_w2, sem_w3):
    wid = lax.axis_index("s") * NC + lax.axis_index("c")
    per_tile = _EP // NW
    nblocks = per_tile // _BA
    nquads = nblocks // 4

    bufs = ((dsti0, srci0, qrows0, krows0, ab0, sem_i0, sem_r0, sem_w0),
            (dsti1, srci1, qrows1, krows1, ab1, sem_i1, sem_r1, sem_w1),
            (dsti2, srci2, qrows2, krows2, ab2, sem_i2, sem_r2, sem_w2),
            (dsti3, srci3, qrows3, krows3, ab3, sem_i3, sem_r3, sem_w3))

    neg = jnp.full((L,), _NEG_BIG, jnp.float32)

    def init(i, c):
        amax_v[pl.ds(i * L, L)] = neg
        return c

    lax.fori_loop(0, (N + L) // L, init, 0)

    base = wid * per_tile
    e16 = lax.iota(jnp.int32, L)

    def idx_issue(b, k):
        d, s2, _, _, _, si, _, _ = bufs[k]
        pltpu.async_copy(dst_h.at[pl.ds(base + b * _BA, _BA)], d, si)
        pltpu.async_copy(src_h.at[pl.ds(base + b * _BA, _BA)], s2, si)

    def idx_wait(k):
        d, s2, _, _, _, si, _, _ = bufs[k]
        pltpu.make_async_copy(dst_h.at[pl.ds(0, _BA)], d, si).wait()
        pltpu.make_async_copy(src_h.at[pl.ds(0, _BA)], s2, si).wait()

    def rows_issue(k):
        d, s2, qr, kr, _, _, sr, _ = bufs[k]
        pltpu.async_copy(qp_h.at[d], qr, sr)
        pltpu.async_copy(kp_h.at[s2], kr, sr)

    def rows_wait(k):
        d, s2, qr, kr, _, _, sr, _ = bufs[k]
        pltpu.make_async_copy(qp_h.at[d], qr, sr).wait()
        pltpu.make_async_copy(kp_h.at[s2], kr, sr).wait()

    for j in range(4):
        idx_issue(j, j)
    idx_wait(0)
    rows_issue(0)
    idx_wait(1)
    rows_issue(1)

    def quad(o, carry):
        for k in range(4):
            d, s2, qr, kr, ab, si, sr, sw = bufs[k]
            b = o * 4 + k
            rows_wait(k)

            @pl.when(o > 0)
            def _():
                pltpu.make_async_copy(ab, alpha_h.at[pl.ds(0, _BA)], sw).wait()

            def grp(g, cc):
                ebase = g * L
                row = e16 + ebase

                def dot(jj, acc):
                    j0 = jj * 4
                    for t in range(4):
                        col = jnp.full((L,), 0, jnp.int32) + (j0 + t)
                        qv = plsc.load_gather(qr, [row, col])
                        kv = plsc.load_gather(kr, [row, col])
                        acc = acc + qv * kv
                    return acc

                acc = lax.fori_loop(0, D // 4, dot, jnp.zeros((L,), jnp.float32))
                acc = acc * _INV_SQRT_D
                ab[pl.ds(ebase, L)] = acc
                d16 = d[pl.ds(ebase, L)]
                cur = plsc.load_gather(amax_v, [d16])
                plsc.store_scatter(amax_v, [d16], jnp.maximum(cur, acc))
                return cc

            lax.fori_loop(0, _BA // L, grp, 0)
            pltpu.async_copy(ab, alpha_h.at[pl.ds(base + b * _BA, _BA)], sw)

            @pl.when(b + 4 < nblocks)
            def _():
                idx_issue(b + 4, k)

            k2 = (k + 2) % 4

            @pl.when(b + 2 < nblocks)
            def _():
                idx_wait(k2)
                rows_issue(k2)

        return carry

    lax.fori_loop(0, nquads, quad, 0)
    pltpu.make_async_copy(ab0, alpha_h.at[pl.ds(0, _BA)], sem_w0).wait()
    pltpu.make_async_copy(ab1, alpha_h.at[pl.ds(0, _BA)], sem_w1).wait()
    pltpu.make_async_copy(ab2, alpha_h.at[pl.ds(0, _BA)], sem_w2).wait()
    pltpu.make_async_copy(ab3, alpha_h.at[pl.ds(0, _BA)], sem_w3).wait()
    pltpu.sync_copy(amax_v.at[pl.ds(0, N)], amaxp_h.at[wid])


def _pass_a(qp, kp, dstp, srcp):
    mesh = plsc.VectorSubcoreMesh(core_axis_name="c", subcore_axis_name="s")
    f = pl.kernel(
        _pass_a_body,
        out_type=(
            jax.ShapeDtypeStruct((_EP,), jnp.float32),
            jax.ShapeDtypeStruct((NW, N), jnp.float32),
        ),
        mesh=mesh,
        compiler_params=pltpu.CompilerParams(
            needs_layout_passes=False, use_tc_tiling_on_sc=False
        ),
        scratch_types=[
            pltpu.VMEM((N + L,), jnp.float32),
            pltpu.VMEM((_BA,), jnp.int32),
            pltpu.VMEM((_BA,), jnp.int32),
            pltpu.VMEM((_BA, 32), jnp.float32),
            pltpu.VMEM((_BA, 32), jnp.float32),
            pltpu.VMEM((_BA,), jnp.float32),
            pltpu.VMEM((_BA,), jnp.int32),
            pltpu.VMEM((_BA,), jnp.int32),
            pltpu.VMEM((_BA, 32), jnp.float32),
            pltpu.VMEM((_BA, 32), jnp.float32),
            pltpu.VMEM((_BA,), jnp.float32),
            pltpu.VMEM((_BA,), jnp.int32),
            pltpu.VMEM((_BA,), jnp.int32),
            pltpu.VMEM((_BA, 32), jnp.float32),
            pltpu.VMEM((_BA, 32), jnp.float32),
            pltpu.VMEM((_BA,), jnp.float32),
            pltpu.VMEM((_BA,), jnp.int32),
            pltpu.VMEM((_BA,), jnp.int32),
            pltpu.VMEM((_BA, 32), jnp.float32),
            pltpu.VMEM((_BA, 32), jnp.float32),
            pltpu.VMEM((_BA,), jnp.float32),
            pltpu.SemaphoreType.DMA,
            pltpu.SemaphoreType.DMA,
            pltpu.SemaphoreType.DMA,
            pltpu.SemaphoreType.DMA,
            pltpu.SemaphoreType.DMA,
            pltpu.SemaphoreType.DMA,
            pltpu.SemaphoreType.DMA,
            pltpu.SemaphoreType.DMA,
            pltpu.SemaphoreType.DMA,
            pltpu.SemaphoreType.DMA,
            pltpu.SemaphoreType.DMA,
            pltpu.SemaphoreType.DMA,
        ],
    )
    return f(qp, kp, dstp, srcp)


_UROWS = 102400


def _pass_b_body(v0_h, v1_h, dst_h, src_h, alpha_h, amax_h, u0_h, u1_h,
                 u_sh, amax_sh, zbuf,
                 dsti0, srci0, alphab0, amb0, vrows0, wbuf0, dscat0,
                 dsti1, srci1, alphab1, amb1, vrows1, wbuf1, dscat1,
                 dsti2, srci2, alphab2, amb2, vrows2, wbuf2, dscat2,
                 dsti3, srci3, alphab3, amb3, vrows3, wbuf3, dscat3,
                 sem_i0, sem_i1, sem_i2, sem_i3,
                 sem_r0, sem_r1, sem_r2, sem_r3,
                 sem_a0, sem_a1, sem_a2, sem_a3,
                 sem_u0, sem_u1, sem_u2, sem_u3):
    c = lax.axis_index("c")
    s = lax.axis_index("s")
    per_tile = _EP // NS
    nblocks = per_tile // _BA
    nquads = nblocks // 4
    rows_per_tile = _UROWS // NS

    bufs = ((dsti0, srci0, alphab0, amb0, vrows0, wbuf0, dscat0,
             sem_i0, sem_r0, sem_a0, sem_u0),
            (dsti1, srci1, alphab1, amb1, vrows1, wbuf1, dscat1,
             sem_i1, sem_r1, sem_a1, sem_u1),
            (dsti2, srci2, alphab2, amb2, vrows2, wbuf2, dscat2,
             sem_i2, sem_r2, sem_a2, sem_u2),
            (dsti3, srci3, alphab3, amb3, vrows3, wbuf3, dscat3,
             sem_i3, sem_r3, sem_a3, sem_u3))

    def zb(r, cc):
        zbuf[r, :] = jnp.zeros((L,), jnp.float32)
        return cc

    lax.fori_loop(0, 128, zb, 0)

    def z(i, cc):
        pltpu.sync_copy(zbuf, u_sh.at[pl.ds(s * rows_per_tile + i * 128, 128), :])
        return cc

    lax.fori_loop(0, rows_per_tile // 128, z, 0)

    @pl.when(s == 0)
    def _():
        pltpu.sync_copy(amax_h, amax_sh.at[pl.ds(0, N)])
        pltpu.sync_copy(zbuf.at[0, :], amax_sh.at[pl.ds(N, L)])

    plsc.subcore_barrier()

    base = s * per_tile
    e16 = lax.iota(jnp.int32, L)

    def idx_issue(b, k):
        d, s2, al, _, _, _, _, si, _, _, _ = bufs[k]
        off = base + b * _BA
        pltpu.async_copy(dst_h.at[pl.ds(off, _BA)], d, si)
        pltpu.async_copy(src_h.at[pl.ds(off, _BA)], s2, si)
        pltpu.async_copy(alpha_h.at[pl.ds(off, _BA)], al, si)

    def idx_wait(k):
        d, s2, al, _, _, _, _, si, _, _, _ = bufs[k]
        pltpu.make_async_copy(dst_h.at[pl.ds(0, _BA)], d, si).wait()
        pltpu.make_async_copy(src_h.at[pl.ds(0, _BA)], s2, si).wait()
        pltpu.make_async_copy(alpha_h.at[pl.ds(0, _BA)], al, si).wait()

    def gath_issue(k):
        d, s2, _, am, vr, _, _, _, sr, sa, _ = bufs[k]
        pltpu.async_copy(amax_sh.at[d], am, sa)

        @pl.when(c == 0)
        def _():
            pltpu.async_copy(v0_h.at[s2], vr, sr)

        @pl.when(c != 0)
        def _():
            pltpu.async_copy(v1_h.at[s2], vr, sr)

    def gath_wait(k):
        d, s2, _, am, vr, _, _, _, sr, sa, _ = bufs[k]
        pltpu.make_async_copy(amax_sh.at[d], am, sa).wait()
        pltpu.make_async_copy(v0_h.at[s2], vr, sr).wait()

    for j in range(4):
        idx_issue(j, j)
    idx_wait(0)
    gath_issue(0)
    idx_wait(1)
    gath_issue(1)

    def quad(o, carry):
        for k in range(4):
            d, s2, al, am, vr, wb, dsc, si, sr, sa, su = bufs[k]
            b = o * 4 + k
            gath_wait(k)

            @pl.when(o > 0)
            def _():
                pltpu.make_async_copy(wb, u_sh.at[d], su).wait()

            def grp(g, gg):
                ebase = g * L
                row = e16 + ebase
                dd = d[pl.ds(ebase, L)]
                dsc[pl.ds(ebase, L)] = dd
                ex = jnp.exp(al[pl.ds(ebase, L)] - am[pl.ds(ebase, L)])

                def wcol(ff, cc2):
                    f0 = ff * 4
                    for t in range(4):
                        col = jnp.full((L,), 0, jnp.int32) + (f0 + t)
                        w = plsc.load_gather(vr, [row, col]) * ex
                        plsc.store_scatter(wb, [row, col], w)
                    return cc2

                lax.fori_loop(0, L // 4, wcol, 0)
                return gg

            lax.fori_loop(0, _BA // L, grp, 0)
            pltpu.async_copy(wb, u_sh.at[dsc], su, add=True)

            @pl.when(b + 4 < nblocks)
            def _():
                idx_issue(b + 4, k)

            k2 = (k + 2) % 4

            @pl.when(b + 2 < nblocks)
            def _():
                idx_wait(k2)
                gath_issue(k2)

        return carry

    lax.fori_loop(0, nquads, quad, 0)
    pltpu.make_async_copy(wbuf0, u_sh.at[dscat0], sem_u0).wait()
    pltpu.make_async_copy(wbuf1, u_sh.at[dscat1], sem_u1).wait()
    pltpu.make_async_copy(wbuf2, u_sh.at[dscat2], sem_u2).wait()
    pltpu.make_async_copy(wbuf3, u_sh.at[dscat3], sem_u3).wait()
    plsc.subcore_barrier()

    @pl.when(c == 0)
    def _():
        pltpu.sync_copy(
            u_sh.at[pl.ds(s * rows_per_tile, rows_per_tile), :],
            u0_h.at[pl.ds(s * rows_per_tile, rows_per_tile), :],
        )

    @pl.when(c != 0)
    def _():
        pltpu.sync_copy(
            u_sh.at[pl.ds(s * rows_per_tile, rows_per_tile), :],
            u1_h.at[pl.ds(s * rows_per_tile, rows_per_tile), :],
        )


def _pass_b(v0p, v1p, dstp, srcp, alpha_p, amax):
    mesh = plsc.VectorSubcoreMesh(core_axis_name="c", subcore_axis_name="s")
    f = pl.kernel(
        _pass_b_body,
        out_type=(
            jax.ShapeDtypeStruct((_UROWS, L), jnp.float32),
            jax.ShapeDtypeStruct((_UROWS, L), jnp.float32),
        ),
        mesh=mesh,
        compiler_params=pltpu.CompilerParams(
            needs_layout_passes=False, use_tc_tiling_on_sc=False
        ),
        scratch_types=[
            pltpu.VMEM_SHARED((_UROWS, L), jnp.float32),
            pltpu.VMEM_SHARED((N + L,), jnp.float32),
            pltpu.VMEM((128, L), jnp.float32),
            pltpu.VMEM((_BA,), jnp.int32),
            pltpu.VMEM((_BA,), jnp.int32),
            pltpu.VMEM((_BA,), jnp.float32),
            pltpu.VMEM((_BA,), jnp.float32),
            pltpu.VMEM((_BA, L), jnp.float32),
            pltpu.VMEM((_BA, L), jnp.float32),
            pltpu.VMEM((_BA,), jnp.int32),
            pltpu.VMEM((_BA,), jnp.int32),
            pltpu.VMEM((_BA,), jnp.int32),
            pltpu.VMEM((_BA,), jnp.float32),
            pltpu.VMEM((_BA,), jnp.float32),
            pltpu.VMEM((_BA, L), jnp.float32),
            pltpu.VMEM((_BA, L), jnp.float32),
            pltpu.VMEM((_BA,), jnp.int32),
            pltpu.VMEM((_BA,), jnp.int32),
            pltpu.VMEM((_BA,), jnp.int32),
            pltpu.VMEM((_BA,), jnp.float32),
            pltpu.VMEM((_BA,), jnp.float32),
            pltpu.VMEM((_BA, L), jnp.float32),
            pltpu.VMEM((_BA, L), jnp.float32),
            pltpu.VMEM((_BA,), jnp.int32),
            pltpu.VMEM((_BA,), jnp.int32),
            pltpu.VMEM((_BA,), jnp.int32),
            pltpu.VMEM((_BA,), jnp.float32),
            pltpu.VMEM((_BA,), jnp.float32),
            pltpu.VMEM((_BA, L), jnp.float32),
            pltpu.VMEM((_BA, L), jnp.float32),
            pltpu.VMEM((_BA,), jnp.int32),
            pltpu.SemaphoreType.DMA,
            pltpu.SemaphoreType.DMA,
            pltpu.SemaphoreType.DMA,
            pltpu.SemaphoreType.DMA,
            pltpu.SemaphoreType.DMA,
            pltpu.SemaphoreType.DMA,
            pltpu.SemaphoreType.DMA,
            pltpu.SemaphoreType.DMA,
            pltpu.SemaphoreType.DMA,
            pltpu.SemaphoreType.DMA,
            pltpu.SemaphoreType.DMA,
            pltpu.SemaphoreType.DMA,
            pltpu.SemaphoreType.DMA,
            pltpu.SemaphoreType.DMA,
            pltpu.SemaphoreType.DMA,
            pltpu.SemaphoreType.DMA,
        ],
    )
    return f(v0p, v1p, dstp, srcp, alpha_p, amax)


def _amax_merge_body(p_ref, o_ref):
    o_ref[...] = jnp.max(p_ref[...], axis=0, keepdims=True)


def _amax_merge(amaxp):
    return pl.pallas_call(
        _amax_merge_body,
        out_shape=jax.ShapeDtypeStruct((1, N), jnp.float32),
    )(amaxp)[0]


_PRJ = 2048


def _project_body(x_ref, wq_ref, wk_ref, w0_ref, w1_ref, ws_ref,
                  qp_ref, kp_ref, v0_ref, v1_ref, s_ref):
    x = x_ref[...]
    dot = lambda w: jnp.dot(x, w[...], preferred_element_type=jnp.float32)
    qp_ref[...] = dot(wq_ref)
    kp_ref[...] = dot(wk_ref)
    v0_ref[...] = dot(w0_ref)
    v1_ref[...] = dot(w1_ref)
    s_ref[...] = dot(ws_ref)


def _project(h_pad, params, i):
    def wpad(W, b, cols, col0=0, one_col=None):
        m = jnp.zeros((D + 1, cols), jnp.float32)
        m = lax.dynamic_update_slice(m, W.T[:, col0:col0 + 12] if cols == L else W.T, (0, 0))
        m = lax.dynamic_update_slice(m, (b[col0:col0 + 12] if cols == L else b)[None, :], (D, 0))
        if one_col is not None:
            m = m.at[D, one_col].set(1.0)
        return m

    Wq = wpad(params["conv%d_Wq" % i], params["conv%d_bq" % i], 32)
    Wk = wpad(params["conv%d_Wk" % i], params["conv%d_bk" % i], 32)
    Wv = params["conv%d_Wv" % i]
    bv = params["conv%d_bv" % i]
    W0 = jnp.zeros((D + 1, L), jnp.float32)
    W0 = lax.dynamic_update_slice(W0, Wv.T[:, :12], (0, 0))
    W0 = lax.dynamic_update_slice(W0, bv[None, :12], (D, 0))
    W0 = W0.at[D, 12].set(1.0)
    W1 = jnp.zeros((D + 1, L), jnp.float32)
    W1 = lax.dynamic_update_slice(W1, Wv.T[:, 12:], (0, 0))
    W1 = lax.dynamic_update_slice(W1, bv[None, 12:], (D, 0))
    Ws = jnp.concatenate(
        [params["conv%d_Ws" % i].T, params["conv%d_bs" % i][None, :]], axis=0
    )
    return pl.pallas_call(
        _project_body,
        grid=(_NPAD // _PRJ,),
        in_specs=[
            pl.BlockSpec((_PRJ, D + 1), lambda j: (j, 0)),
            pl.BlockSpec((D + 1, 32), lambda j: (0, 0)),
            pl.BlockSpec((D + 1, 32), lambda j: (0, 0)),
            pl.BlockSpec((D + 1, L), lambda j: (0, 0)),
            pl.BlockSpec((D + 1, L), lambda j: (0, 0)),
            pl.BlockSpec((D + 1, D), lambda j: (0, 0)),
        ],
        out_specs=[
            pl.BlockSpec((_PRJ, 32), lambda j: (j, 0)),
            pl.BlockSpec((_PRJ, 32), lambda j: (j, 0)),
            pl.BlockSpec((_PRJ, L), lambda j: (j, 0)),
            pl.BlockSpec((_PRJ, L), lambda j: (j, 0)),
            pl.BlockSpec((_PRJ, D), lambda j: (j, 0)),
        ],
        out_shape=[
            jax.ShapeDtypeStruct((_NPAD, 32), jnp.float32),
            jax.ShapeDtypeStruct((_NPAD, 32), jnp.float32),
            jax.ShapeDtypeStruct((_NPAD, L), jnp.float32),
            jax.ShapeDtypeStruct((_NPAD, L), jnp.float32),
            jax.ShapeDtypeStruct((_NPAD, D), jnp.float32),
        ],
    )(h_pad, Wq, Wk, W0, W1, Ws)


def _combine_body(u0_ref, u1_ref, s_ref, h_ref):
    u0 = u0_ref[...]
    u1 = u1_ref[...]
    den = u0[:, 12:13] + 1e-16
    num = jnp.concatenate([u0[:, :12], u1[:, :12]], axis=1)
    h = jnp.maximum(num / den + s_ref[...], 0.0)
    ones = jnp.ones((h.shape[0], 1), jnp.float32)
    h_ref[...] = jnp.concatenate([h, ones], axis=1)


def _combine(u0, u1, s_pad):
    return pl.pallas_call(
        _combine_body,
        grid=(_NPAD // _PRJ,),
        in_specs=[
            pl.BlockSpec((_PRJ, L), lambda j: (j, 0)),
            pl.BlockSpec((_PRJ, L), lambda j: (j, 0)),
            pl.BlockSpec((_PRJ, D), lambda j: (j, 0)),
        ],
        out_specs=pl.BlockSpec((_PRJ, D + 1), lambda j: (j, 0)),
        out_shape=jax.ShapeDtypeStruct((_NPAD, D + 1), jnp.float32),
    )(u0, u1, s_pad)


_PR = 2048
_NPAD = 102400


def _pool_body(b_ref, h_ref, o_ref):
    i = pl.program_id(0)

    @pl.when(i == 0)
    def _():
        o_ref[...] = jnp.zeros_like(o_ref)

    ids = b_ref[0, :]
    onehot = (ids[:, None] == lax.broadcasted_iota(jnp.int32, (1, G), 1)).astype(
        jnp.float32
    )
    o_ref[...] += lax.dot_general(
        onehot,
        h_ref[...],
        (((0,), (0,)), ((), ())),
        preferred_element_type=jnp.float32,
    )


def _pool(h_pad, batch):
    bpad = jnp.full((1, _NPAD), G, jnp.int32)
    bpad = lax.dynamic_update_slice(bpad, batch[None, :], (0, 0))
    h_aug = jnp.pad(h_pad, ((0, 0), (0, 32 - (D + 1))))
    return pl.pallas_call(
        _pool_body,
        grid=(_NPAD // _PR,),
        in_specs=[
            pl.BlockSpec((1, _PR), lambda i: (0, i)),
            pl.BlockSpec((_PR, 32), lambda i: (i, 0)),
        ],
        out_specs=pl.BlockSpec((G, 32), lambda i: (0, 0)),
        out_shape=jax.ShapeDtypeStruct((G, 32), jnp.float32),
    )(bpad, h_aug)


def _pad_rows(a, cols=32, extra_rows=L):
    n, d = a.shape
    out = jnp.zeros((n + extra_rows, cols), jnp.float32)
    return lax.dynamic_update_slice(out, a, (0, 0))


def kernel(x, edge_index, batch, global_features, params):
    src = edge_index[0]
    dst = edge_index[1]
    pad = jnp.full((_EP - E,), N, jnp.int32)
    srcp = jnp.concatenate([src, pad])
    dstp = jnp.concatenate([dst, pad])

    h_pad = jnp.zeros((_NPAD, D + 1), jnp.float32)
    h_pad = lax.dynamic_update_slice(h_pad, x, (0, 0))
    h_pad = h_pad.at[:, D].set(1.0)
    for i in range(NL):
        qp, kp, v0p, v1p, s_pad = _project(h_pad, params, i)
        alpha_p, amaxp = _pass_a(qp, kp, dstp, srcp)
        amax = _amax_merge(amaxp)
        u0, u1 = _pass_b(v0p, v1p, dstp, srcp, alpha_p, amax)
        h_pad = _combine(u0, u1, s_pad)

    pooled_aug = _pool(h_pad, batch)
    pooled = pooled_aug[:, :D] / jnp.maximum(pooled_aug[:, D:D + 1], 1.0)
    z = _linear(pooled, params["lin1_W"], params["lin1_b"], relu=True)
    z = _linear(z, params["lin2_W"], params["lin2_b"], relu=True)
    z = _linear(z, params["lin3_W"], params["lin3_b"])
    return jnp.squeeze(z, axis=-1)
